# Initial kernel scaffold; baseline (speedup 1.0000x reference)
import jax
import jax.numpy as jnp
from jax.experimental import pallas as pl


def kernel(x, edge_index, edge_weight, W_first, b_first, W_rel, b_rel, fuse_weight, W_out, b_out):
    raise NotImplementedError("write your pallas kernel here")


# speedup vs baseline: 7.9278x; 7.9278x over previous
import functools

import jax
import jax.numpy as jnp
from jax import lax
from jax.experimental import pallas as pl
from jax.experimental.pallas import tpu as pltpu
from jax.experimental.pallas import tpu_sc as plsc

N = 100000
E = 1600000
FEAT = 128
HID = 32
HALF = 16
MAX_DEG = 5

LANES = 128
ROWS = 8
CHUNK = ROWS * LANES
NQ = 98
EPT = NQ * CHUNK
EPAD = 16 * EPT
ROWS_PER_TILE = 6256
NPAD = 16 * ROWS_PER_TILE

CW = 8
RBLK = 2000
GRID = N // RBLK


def _sc_seg_body(src3, sel3, table, zrow_f, out, idx_v, sel_v, rows_v, acc,
                 gsem):
    c = lax.axis_index("c")
    s = lax.axis_index("s")
    base = s * ROWS_PER_TILE

    pltpu.sync_copy(zrow_f, acc.at[pl.ds(base, ROWS_PER_TILE)])
    plsc.subcore_barrier()

    def body(q, carry):
        pltpu.sync_copy(src3.at[c, s, q], idx_v)
        pltpu.sync_copy(sel3.at[s, q], sel_v)
        got = [
            pltpu.async_copy(table.at[idx_v.at[j]],
                             rows_v.at[pl.ds(j * LANES, LANES)], gsem)
            for j in range(ROWS)
        ]
        for g in got:
            g.wait()
        for j in range(ROWS):
            pltpu.sync_copy(rows_v.at[pl.ds(j * LANES, LANES)],
                            acc.at[sel_v.at[j]], add=True)
        return carry

    lax.fori_loop(0, NQ, body, 0)
    plsc.subcore_barrier()

    pltpu.sync_copy(acc.at[pl.ds(base, ROWS_PER_TILE)],
                    out.at[c, pl.ds(base, ROWS_PER_TILE)])


@functools.cache
def _sc_segsum_kernel():
  return pl.kernel(
    _sc_seg_body,
    out_type=jax.ShapeDtypeStruct((2, NPAD, HALF), jnp.float32),
    mesh=plsc.VectorSubcoreMesh(core_axis_name="c", subcore_axis_name="s"),
    scratch_types=[
        pltpu.VMEM((ROWS, LANES), jnp.int32),
        pltpu.VMEM((ROWS, LANES), jnp.int32),
        pltpu.VMEM((CHUNK, HALF), jnp.float32),
        pltpu.VMEM_SHARED((NPAD, HALF), jnp.float32),
        pltpu.SemaphoreType.DMA,
    ],
    compiler_params=pltpu.CompilerParams(use_tc_tiling_on_sc=False),
  )


def _sc_cnt_body(sel3, zrow_i, ones_h, cnt_out, sel_v, ones_v, cnt):
    c = lax.axis_index("c")
    s = lax.axis_index("s")
    base = s * ROWS_PER_TILE
    pltpu.sync_copy(zrow_i, cnt.at[pl.ds(base, ROWS_PER_TILE)])
    pltpu.sync_copy(ones_h, ones_v)
    plsc.subcore_barrier()

    def body(q, carry):
        pltpu.sync_copy(sel3.at[s, q], sel_v)
        for j in range(ROWS):
            pltpu.sync_copy(ones_v, cnt.at[sel_v.at[j]], add=True)
        return carry

    q0 = c * (NQ // 2)
    lax.fori_loop(q0, q0 + NQ // 2, body, 0)
    plsc.subcore_barrier()

    pltpu.sync_copy(cnt.at[pl.ds(base, ROWS_PER_TILE)],
                    cnt_out.at[c, pl.ds(base, ROWS_PER_TILE)])


@functools.cache
def _sc_count_kernel():
  return pl.kernel(
    _sc_cnt_body,
    out_type=jax.ShapeDtypeStruct((2, NPAD, CW), jnp.int32),
    mesh=plsc.VectorSubcoreMesh(core_axis_name="c", subcore_axis_name="s"),
    scratch_types=[
        pltpu.VMEM((ROWS, LANES), jnp.int32),
        pltpu.VMEM((LANES, CW), jnp.int32),
        pltpu.VMEM_SHARED((NPAD, CW), jnp.int32),
    ],
    compiler_params=pltpu.CompilerParams(use_tc_tiling_on_sc=False),
  )


def _sc_segsum(src3, sel3, table, zrow_f):
    return _sc_segsum_kernel()(src3, sel3, table, zrow_f)


def _sc_count(sel3, zrow_i, ones_h):
    return _sc_count_kernel()(sel3, zrow_i, ones_h)


def _tc_first_body(x_ref, w_ref, b_ref, o_ref):
    acc = lax.dot_general(x_ref[...], w_ref[...],
                          (((1,), (1,)), ((), ())),
                          preferred_element_type=jnp.float32)
    o_ref[...] = jnp.maximum(acc + b_ref[...], 0.0)


def _tc_first(x, w, b):
    return pl.pallas_call(
        _tc_first_body,
        grid=(GRID,),
        in_specs=[
            pl.BlockSpec((RBLK, FEAT), lambda i: (i, 0)),
            pl.BlockSpec((HID, FEAT), lambda i: (0, 0)),
            pl.BlockSpec((1, HID), lambda i: (0, 0)),
        ],
        out_specs=pl.BlockSpec((RBLK, HID), lambda i: (i, 0)),
        out_shape=jax.ShapeDtypeStruct((N, HID), jnp.float32),
    )(x, w, b.reshape(1, HID))


def _deg_select(agg0, agg1, cur, cnt, w_ref, b_ref):
    hagg = jnp.concatenate([agg0, agg1], axis=1) + cur
    deg = jnp.minimum(cnt, MAX_DEG)
    acc = jnp.zeros((RBLK, HID), jnp.float32)
    for k in range(MAX_DEG + 1):
        ok = lax.dot_general(hagg, w_ref[k], (((1,), (1,)), ((), ())),
                             preferred_element_type=jnp.float32) + b_ref[k]
        acc = acc + jnp.where(deg == k, ok, 0.0)
    return acc


def _tc_mid_body(a0_ref, a1_ref, cur_ref, xf_ref, cnt_ref, w_ref, b_ref,
                 fw_ref, o_ref):
    acc = _deg_select(a0_ref[...], a1_ref[...], cur_ref[...], cnt_ref[...],
                      w_ref, b_ref)
    o_ref[...] = acc + fw_ref[0, 0] * xf_ref[...]


def _tc_final_body(a0_ref, a1_ref, cur_ref, xf_ref, cnt_ref, w_ref, b_ref,
                   fw_ref, wo_ref, bo_ref, o_ref):
    acc = _deg_select(a0_ref[...], a1_ref[...], cur_ref[...], cnt_ref[...],
                      w_ref, b_ref)
    o = acc + fw_ref[0, 0] * xf_ref[...]
    l0 = jnp.sum(o * wo_ref[0], axis=1, keepdims=True) + bo_ref[0, 0]
    l1 = jnp.sum(o * wo_ref[1], axis=1, keepdims=True) + bo_ref[0, 1]
    m = jnp.maximum(l0, l1)
    lse = m + jnp.log(jnp.exp(l0 - m) + jnp.exp(l1 - m))
    o_ref[...] = jnp.concatenate([l0 - lse, l1 - lse], axis=1)


_ROW_SPECS = [
    pl.BlockSpec((RBLK, HALF), lambda i: (i, 0)),
    pl.BlockSpec((RBLK, HALF), lambda i: (i, 0)),
    pl.BlockSpec((RBLK, HID), lambda i: (i, 0)),
    pl.BlockSpec((RBLK, HID), lambda i: (i, 0)),
    pl.BlockSpec((RBLK, 1), lambda i: (i, 0)),
    pl.BlockSpec((MAX_DEG + 1, HID, HID), lambda i: (0, 0, 0)),
    pl.BlockSpec((MAX_DEG + 1, 1, HID), lambda i: (0, 0, 0)),
    pl.BlockSpec((1, 1), lambda i: (0, 0)),
]


def _tc_mid(a0, a1, cur, xf, cnt, w, b, fw):
    return pl.pallas_call(
        _tc_mid_body,
        grid=(GRID,),
        in_specs=_ROW_SPECS,
        out_specs=pl.BlockSpec((RBLK, HID), lambda i: (i, 0)),
        out_shape=jax.ShapeDtypeStruct((N, HID), jnp.float32),
    )(a0, a1, cur, xf, cnt, w, b.reshape(MAX_DEG + 1, 1, HID),
      fw.reshape(1, 1))


def _tc_final(a0, a1, cur, xf, cnt, w, b, fw, wo, bo):
    return pl.pallas_call(
        _tc_final_body,
        grid=(GRID,),
        in_specs=_ROW_SPECS + [
            pl.BlockSpec((2, HID), lambda i: (0, 0)),
            pl.BlockSpec((1, 2), lambda i: (0, 0)),
        ],
        out_specs=pl.BlockSpec((RBLK, 2), lambda i: (i, 0)),
        out_shape=jax.ShapeDtypeStruct((N, 2), jnp.float32),
    )(a0, a1, cur, xf, cnt, w, b.reshape(MAX_DEG + 1, 1, HID),
      fw.reshape(1, 1), wo, bo.reshape(1, 2))


def kernel(x, edge_index, edge_weight, W_first, b_first, W_rel, b_rel,
           fuse_weight, W_out, b_out):
    src = edge_index[0].astype(jnp.int32)
    dst = edge_index[1].astype(jnp.int32)
    sel = jnp.where(src == dst, jnp.int32(N), dst)
    src_p = jnp.concatenate([src, jnp.zeros((EPAD - E,), jnp.int32)])
    sel_p = jnp.concatenate([sel, jnp.full((EPAD - E,), N, jnp.int32)])
    src3 = jnp.stack([src_p, src_p + N]).reshape(2, 16, NQ, ROWS, LANES)
    sel3 = sel_p.reshape(16, NQ, ROWS, LANES)

    zrow_f = jnp.zeros((ROWS_PER_TILE, HALF), jnp.float32)
    zrow_i = jnp.zeros((ROWS_PER_TILE, CW), jnp.int32)
    ones_h = jnp.ones((LANES, CW), jnp.int32)

    h = _tc_first(x, W_first, b_first)

    cnt = _sc_count(sel3, zrow_i, ones_h)
    cnt2 = (cnt[0, :N, 0] + cnt[1, :N, 0]).reshape(N, 1)

    table1 = jnp.concatenate([h[:, :HALF], h[:, HALF:]], axis=0)
    agg1 = _sc_segsum(src3, sel3, table1, zrow_f)

    cur1 = _tc_mid(agg1[0, :N], agg1[1, :N], h, h, cnt2,
                   W_rel[0], b_rel[0], fuse_weight[0])

    table2 = jnp.concatenate([cur1[:, :HALF], cur1[:, HALF:]], axis=0)
    agg2 = _sc_segsum(src3, sel3, table2, zrow_f)

    return _tc_final(agg2[0, :N], agg2[1, :N], cur1, h, cnt2,
                     W_rel[1], b_rel[1], fuse_weight[1], W_out, b_out)
